# baseline (speedup 1.0000x reference)
import functools

import jax
import jax.numpy as jnp
from jax import lax
from jax.experimental import pallas as pl
from jax.experimental.pallas import tpu as pltpu
from jax.experimental.pallas import tpu_sc as plsc

T = 2048
D = 1024
F = 512
E = 8
BT = 128
P = T * 2 + E * BT
NB = P // BT
NBPAD = 48

NC, NS = 2, 16
NW = NC * NS
TW = T // NW
NG = T // 16
GW = NG // NW
CCH = 16
NEG = -3.0e38

_mesh = plsc.VectorSubcoreMesh(core_axis_name="c", subcore_axis_name="s")
_sc_params = pltpu.CompilerParams(needs_layout_passes=False)


@functools.partial(
    pl.kernel,
    out_type=[
        jax.ShapeDtypeStruct((P, D), jnp.float32),
        jax.ShapeDtypeStruct((NBPAD,), jnp.int32),
        jax.ShapeDtypeStruct((T,), jnp.int32),
        jax.ShapeDtypeStruct((T,), jnp.int32),
        jax.ShapeDtypeStruct((T,), jnp.float32),
        jax.ShapeDtypeStruct((T,), jnp.float32),
    ],
    mesh=_mesh,
    scratch_types=[
        pltpu.VMEM((E * T,), jnp.float32),
        pltpu.VMEM((T,), jnp.int32),
        pltpu.VMEM((T,), jnp.int32),
        pltpu.VMEM((TW,), jnp.int32),
        pltpu.VMEM((TW,), jnp.int32),
        pltpu.VMEM((TW,), jnp.float32),
        pltpu.VMEM((TW,), jnp.float32),
        pltpu.VMEM((NBPAD,), jnp.int32),
        pltpu.VMEM((TW, D), jnp.float32),
        pltpu.SemaphoreType.DMA,
        pltpu.SemaphoreType.DMA,
    ],
    compiler_params=_sc_params,
)
def _route_dispatch(logits_hbm, hidden_hbm,
                    xg_hbm, be_hbm, pos1_hbm, pos2_hbm, w1_hbm, w2_hbm,
                    lt_v, e1_v, e2_v, p1s_v, p2s_v, w1s_v, w2s_v,
                    be_v, rows_v, sem, sem_h):
    cid = lax.axis_index("c")
    sid = lax.axis_index("s")
    wid = sid * NC + cid
    own_lo = wid * GW
    lanes = lax.broadcasted_iota(jnp.int32, (16,), 0)
    zeros16 = jnp.zeros((16,), jnp.int32)

    hid_cp = pltpu.async_copy(hidden_hbm.at[pl.ds(wid * TW, TW)], rows_v, sem_h)
    pltpu.sync_copy(logits_hbm, lt_v)

    def scan_a(g, carry):
        hist, pref = carry
        base = g * 16
        le = [lt_v[pl.ds(e * T + base, 16)] for e in range(E)]
        m1 = le[0]
        for e in range(1, E):
            m1 = jnp.maximum(m1, le[e])
        i1 = jnp.full((16,), -1, jnp.int32)
        for e in range(E):
            i1 = jnp.where((le[e] == m1) & (i1 < 0), e, i1)
        m2 = jnp.full((16,), NEG, jnp.float32)
        l2 = []
        for e in range(E):
            v = jnp.where(i1 == e, NEG, le[e])
            l2.append(v)
            m2 = jnp.maximum(m2, v)
        i2 = jnp.full((16,), -1, jnp.int32)
        for e in range(E):
            i2 = jnp.where((l2[e] == m2) & (i2 < 0), e, i2)
        e1_v[pl.ds(base, 16)] = i1
        e2_v[pl.ds(base, 16)] = i2
        delta = zeros16
        for e in range(E):
            c = (plsc.all_reduce_population_count(i1 == e)
                 + plsc.all_reduce_population_count(i2 == e))
            delta = delta + jnp.where(lanes == e, c, 0)
        hist = hist + delta
        pref = pref + jnp.where(g < own_lo, delta, zeros16)
        return hist, pref
    hist, pref = lax.fori_loop(0, NG, scan_a, (zeros16, zeros16))

    starts = []
    nexts = []
    acc = jnp.int32(0)
    for e in range(E):
        starts.append(acc)
        acc = (acc + hist[e] + (BT - 1)) & jnp.int32(~(BT - 1))
        nexts.append(acc)

    for vb in range(NBPAD // 16):
        bb = (vb * 16 + lanes) * BT
        bev = zeros16
        for e in range(E - 1):
            bev = bev + jnp.where(bb >= nexts[e], 1, 0)
        be_v[pl.ds(vb * 16, 16)] = bev

    @pl.when(wid == 0)
    def _():
        pltpu.sync_copy(be_v, be_hbm)

    runs = [starts[e] + pref[e] + zeros16 for e in range(E)]
    for k in range(GW):
        base = (own_lo + k) * 16
        le = [lt_v[pl.ds(e * T + base, 16)] for e in range(E)]
        i1 = e1_v[pl.ds(base, 16)]
        i2 = e2_v[pl.ds(base, 16)]
        m1 = jnp.full((16,), NEG, jnp.float32)
        m2 = jnp.full((16,), NEG, jnp.float32)
        for e in range(E):
            m1 = jnp.where(i1 == e, le[e], m1)
            m2 = jnp.where(i2 == e, le[e], m2)
        wa = 1.0 / (1.0 + jnp.exp(m2 - m1))
        w1s_v[pl.ds(k * 16, 16)] = wa
        w2s_v[pl.ds(k * 16, 16)] = 1.0 - wa
        p1vec = zeros16
        p2vec = zeros16
        for e in range(E):
            m = i1 == e
            cs = plsc.cumsum(m.astype(jnp.int32))
            p1vec = jnp.where(m, runs[e] + cs - 1, p1vec)
            runs[e] = runs[e] + plsc.all_reduce_population_count(m)
            m = i2 == e
            cs = plsc.cumsum(m.astype(jnp.int32))
            p2vec = jnp.where(m, runs[e] + cs - 1, p2vec)
            runs[e] = runs[e] + plsc.all_reduce_population_count(m)
        p1s_v[pl.ds(k * 16, 16)] = p1vec
        p2s_v[pl.ds(k * 16, 16)] = p2vec

    tb = wid * TW
    pltpu.sync_copy(p1s_v, pos1_hbm.at[pl.ds(tb, TW)])
    pltpu.sync_copy(p2s_v, pos2_hbm.at[pl.ds(tb, TW)])
    pltpu.sync_copy(w1s_v, w1_hbm.at[pl.ds(tb, TW)])
    pltpu.sync_copy(w2s_v, w2_hbm.at[pl.ds(tb, TW)])

    hid_cp.wait()
    c1 = pltpu.async_copy(rows_v, xg_hbm.at[p1s_v], sem)
    c2 = pltpu.async_copy(rows_v, xg_hbm.at[p2s_v], sem)
    c1.wait()
    c2.wait()


def _ffn_body(be_ref, x_ref, wg_ref, wu_ref, wd_ref, y_ref):
    x = x_ref[...].astype(jnp.bfloat16)
    g = lax.dot_general(x, wg_ref[0], (((1,), (1,)), ((), ())),
                        preferred_element_type=jnp.float32)
    u = lax.dot_general(x, wu_ref[0], (((1,), (1,)), ((), ())),
                        preferred_element_type=jnp.float32)
    h = (g * jax.nn.sigmoid(g) * u).astype(jnp.bfloat16)
    y_ref[...] = lax.dot_general(h, wd_ref[0], (((1,), (1,)), ((), ())),
                                 preferred_element_type=jnp.float32)


def _expert_ffn(be, xg, Wg, Wu, Wd):
    grid_spec = pltpu.PrefetchScalarGridSpec(
        num_scalar_prefetch=1,
        grid=(NB,),
        in_specs=[
            pl.BlockSpec((BT, D), lambda i, be: (i, 0)),
            pl.BlockSpec((1, F, D), lambda i, be: (be[i], 0, 0)),
            pl.BlockSpec((1, F, D), lambda i, be: (be[i], 0, 0)),
            pl.BlockSpec((1, D, F), lambda i, be: (be[i], 0, 0)),
        ],
        out_specs=pl.BlockSpec((BT, D), lambda i, be: (i, 0)),
    )
    return pl.pallas_call(
        _ffn_body,
        grid_spec=grid_spec,
        out_shape=jax.ShapeDtypeStruct((P, D), jnp.float32),
    )(be, xg, Wg, Wu, Wd)


@functools.partial(
    pl.kernel,
    out_type=jax.ShapeDtypeStruct((T, D), jnp.float32),
    mesh=_mesh,
    scratch_types=[
        pltpu.VMEM((TW,), jnp.int32),
        pltpu.VMEM((TW,), jnp.int32),
        pltpu.VMEM((TW,), jnp.float32),
        pltpu.VMEM((TW,), jnp.float32),
        pltpu.VMEM((CCH, D), jnp.float32),
        pltpu.VMEM((CCH, D), jnp.float32),
        pltpu.VMEM((CCH, D), jnp.float32),
        pltpu.VMEM((CCH, D), jnp.float32),
        pltpu.VMEM((CCH, D), jnp.float32),
        pltpu.SemaphoreType.DMA,
        pltpu.SemaphoreType.DMA,
    ],
    compiler_params=_sc_params,
)
def _combine(y_hbm, pos1_hbm, pos2_hbm, w1_hbm, w2_hbm, out_hbm,
             p1_v, p2_v, w1_v, w2_v, r1a_v, r2a_v, r1b_v, r2b_v, out_v,
             sem0, sem1):
    cid = lax.axis_index("c")
    sid = lax.axis_index("s")
    wid = sid * NC + cid
    tb = wid * TW
    pltpu.sync_copy(pos1_hbm.at[pl.ds(tb, TW)], p1_v)
    pltpu.sync_copy(pos2_hbm.at[pl.ds(tb, TW)], p2_v)
    pltpu.sync_copy(w1_hbm.at[pl.ds(tb, TW)], w1_v)
    pltpu.sync_copy(w2_hbm.at[pl.ds(tb, TW)], w2_v)

    nch = TW // CCH
    r1 = [r1a_v, r1b_v]
    r2 = [r2a_v, r2b_v]
    sems = [sem0, sem1]

    def fire(ch, slot):
        a = pltpu.async_copy(y_hbm.at[p1_v.at[pl.ds(ch * CCH, CCH)]],
                             r1[slot], sems[slot])
        b = pltpu.async_copy(y_hbm.at[p2_v.at[pl.ds(ch * CCH, CCH)]],
                             r2[slot], sems[slot])
        return a, b

    pend = fire(0, 0)
    for ch in range(nch):
        slot = ch % 2
        pend[0].wait()
        pend[1].wait()
        if ch + 1 < nch:
            pend = fire(ch + 1, (ch + 1) % 2)
        wa = w1_v[pl.ds(ch * CCH, CCH)]
        wb = w2_v[pl.ds(ch * CCH, CCH)]

        def strip(j, _):
            for t in range(CCH):
                out_v[t, pl.ds(j * 16, 16)] = (
                    wa[t] * r1[slot][t, pl.ds(j * 16, 16)]
                    + wb[t] * r2[slot][t, pl.ds(j * 16, 16)])
            return 0
        lax.fori_loop(0, D // 16, strip, 0)
        pltpu.sync_copy(out_v, out_hbm.at[pl.ds(tb + ch * CCH, CCH)])


def kernel(hidden_states, router_logits, Wg, Wu, Wd):
    logits_t = router_logits.T.reshape(E * T)
    xg, be, pos1, pos2, w1, w2 = _route_dispatch(logits_t, hidden_states)
    y = _expert_ffn(be, xg, Wg.astype(jnp.bfloat16), Wu.astype(jnp.bfloat16),
                    Wd.astype(jnp.bfloat16))
    return _combine(y, pos1, pos2, w1, w2)


# speedup vs baseline: 1.0574x; 1.0574x over previous
import functools

import jax
import jax.numpy as jnp
from jax import lax
from jax.experimental import pallas as pl
from jax.experimental.pallas import tpu as pltpu
from jax.experimental.pallas import tpu_sc as plsc

T = 2048
D = 1024
F = 512
E = 8
BT = 128
P = T * 2 + E * BT
NB = P // BT
NBPAD = 48

NC, NS = 2, 16
NW = NC * NS
TW = T // NW
NG = T // 16
GW = NG // NW
CCH = 16
NEG = -3.0e38

_mesh = plsc.VectorSubcoreMesh(core_axis_name="c", subcore_axis_name="s")
_sc_params = pltpu.CompilerParams(needs_layout_passes=False)


@functools.partial(
    pl.kernel,
    out_type=[
        jax.ShapeDtypeStruct((P, D), jnp.float32),
        jax.ShapeDtypeStruct((NBPAD,), jnp.int32),
        jax.ShapeDtypeStruct((T,), jnp.int32),
        jax.ShapeDtypeStruct((T,), jnp.int32),
        jax.ShapeDtypeStruct((T,), jnp.float32),
        jax.ShapeDtypeStruct((T,), jnp.float32),
    ],
    mesh=_mesh,
    scratch_types=[
        pltpu.VMEM((E * T,), jnp.float32),
        pltpu.VMEM((T,), jnp.int32),
        pltpu.VMEM((T,), jnp.int32),
        pltpu.VMEM((TW,), jnp.int32),
        pltpu.VMEM((TW,), jnp.int32),
        pltpu.VMEM((TW,), jnp.float32),
        pltpu.VMEM((TW,), jnp.float32),
        pltpu.VMEM((NBPAD,), jnp.int32),
        pltpu.VMEM((TW, D), jnp.float32),
        pltpu.SemaphoreType.DMA,
        pltpu.SemaphoreType.DMA,
    ],
    compiler_params=_sc_params,
)
def _route_dispatch(logits_hbm, hidden_hbm,
                    xg_hbm, be_hbm, pos1_hbm, pos2_hbm, w1_hbm, w2_hbm,
                    lt_v, e1_v, e2_v, p1s_v, p2s_v, w1s_v, w2s_v,
                    be_v, rows_v, sem, sem_h):
    cid = lax.axis_index("c")
    sid = lax.axis_index("s")
    wid = sid * NC + cid
    own_lo = wid * GW
    lanes = lax.broadcasted_iota(jnp.int32, (16,), 0)
    zeros16 = jnp.zeros((16,), jnp.int32)

    hid_cp = pltpu.async_copy(hidden_hbm.at[pl.ds(wid * TW, TW)], rows_v, sem_h)
    pltpu.sync_copy(logits_hbm, lt_v)

    def scan_a(g, carry):
        hist, pref = carry
        base = g * 16
        le = [lt_v[pl.ds(e * T + base, 16)] for e in range(E)]
        m1 = le[0]
        for e in range(1, E):
            m1 = jnp.maximum(m1, le[e])
        i1 = jnp.full((16,), -1, jnp.int32)
        for e in range(E):
            i1 = jnp.where((le[e] == m1) & (i1 < 0), e, i1)
        m2 = jnp.full((16,), NEG, jnp.float32)
        l2 = []
        for e in range(E):
            v = jnp.where(i1 == e, NEG, le[e])
            l2.append(v)
            m2 = jnp.maximum(m2, v)
        i2 = jnp.full((16,), -1, jnp.int32)
        for e in range(E):
            i2 = jnp.where((l2[e] == m2) & (i2 < 0), e, i2)
        e1_v[pl.ds(base, 16)] = i1
        e2_v[pl.ds(base, 16)] = i2
        delta = zeros16
        for e in range(E):
            c = (plsc.all_reduce_population_count(i1 == e)
                 + plsc.all_reduce_population_count(i2 == e))
            delta = delta + jnp.where(lanes == e, c, 0)
        hist = hist + delta
        pref = pref + jnp.where(g < own_lo, delta, zeros16)
        return hist, pref
    hist, pref = lax.fori_loop(0, NG, scan_a, (zeros16, zeros16))

    starts = []
    nexts = []
    acc = jnp.int32(0)
    for e in range(E):
        starts.append(acc)
        acc = (acc + hist[e] + (BT - 1)) & jnp.int32(~(BT - 1))
        nexts.append(acc)

    for vb in range(NBPAD // 16):
        bb = (vb * 16 + lanes) * BT
        bev = zeros16
        for e in range(E - 1):
            bev = bev + jnp.where(bb >= nexts[e], 1, 0)
        be_v[pl.ds(vb * 16, 16)] = bev

    @pl.when(wid == 0)
    def _():
        pltpu.sync_copy(be_v, be_hbm)

    runs = [starts[e] + pref[e] + zeros16 for e in range(E)]
    for k in range(GW):
        base = (own_lo + k) * 16
        le = [lt_v[pl.ds(e * T + base, 16)] for e in range(E)]
        i1 = e1_v[pl.ds(base, 16)]
        i2 = e2_v[pl.ds(base, 16)]
        m1 = jnp.full((16,), NEG, jnp.float32)
        m2 = jnp.full((16,), NEG, jnp.float32)
        for e in range(E):
            m1 = jnp.where(i1 == e, le[e], m1)
            m2 = jnp.where(i2 == e, le[e], m2)
        wa = 1.0 / (1.0 + jnp.exp(m2 - m1))
        w1s_v[pl.ds(k * 16, 16)] = wa
        w2s_v[pl.ds(k * 16, 16)] = 1.0 - wa
        p1vec = zeros16
        p2vec = zeros16
        for e in range(E):
            m = i1 == e
            cs = plsc.cumsum(m.astype(jnp.int32))
            p1vec = jnp.where(m, runs[e] + cs - 1, p1vec)
            runs[e] = runs[e] + plsc.all_reduce_population_count(m)
            m = i2 == e
            cs = plsc.cumsum(m.astype(jnp.int32))
            p2vec = jnp.where(m, runs[e] + cs - 1, p2vec)
            runs[e] = runs[e] + plsc.all_reduce_population_count(m)
        p1s_v[pl.ds(k * 16, 16)] = p1vec
        p2s_v[pl.ds(k * 16, 16)] = p2vec

    tb = wid * TW
    pltpu.sync_copy(p1s_v, pos1_hbm.at[pl.ds(tb, TW)])
    pltpu.sync_copy(p2s_v, pos2_hbm.at[pl.ds(tb, TW)])
    pltpu.sync_copy(w1s_v, w1_hbm.at[pl.ds(tb, TW)])
    pltpu.sync_copy(w2s_v, w2_hbm.at[pl.ds(tb, TW)])

    hid_cp.wait()
    c1 = pltpu.async_copy(rows_v, xg_hbm.at[p1s_v], sem)
    c2 = pltpu.async_copy(rows_v, xg_hbm.at[p2s_v], sem)
    c1.wait()
    c2.wait()


def _ffn_body(be_ref, x_ref, wg_ref, wu_ref, wd_ref, y_ref,
              wgb_ref, wub_ref, wdb_ref, flag_ref):
    i = pl.program_id(0)
    e = be_ref[i]

    @pl.when((i == 0) | (e != flag_ref[0]))
    def _():
        wgb_ref[...] = wg_ref[0].astype(jnp.bfloat16)
        wub_ref[...] = wu_ref[0].astype(jnp.bfloat16)
        wdb_ref[...] = wd_ref[0].astype(jnp.bfloat16)
        flag_ref[0] = e

    x = x_ref[...].astype(jnp.bfloat16)
    g = lax.dot_general(x, wgb_ref[...], (((1,), (1,)), ((), ())),
                        preferred_element_type=jnp.float32)
    u = lax.dot_general(x, wub_ref[...], (((1,), (1,)), ((), ())),
                        preferred_element_type=jnp.float32)
    h = (g * jax.nn.sigmoid(g) * u).astype(jnp.bfloat16)
    y_ref[...] = lax.dot_general(h, wdb_ref[...], (((1,), (1,)), ((), ())),
                                 preferred_element_type=jnp.float32)


def _expert_ffn(be, xg, Wg, Wu, Wd):
    grid_spec = pltpu.PrefetchScalarGridSpec(
        num_scalar_prefetch=1,
        grid=(NB,),
        in_specs=[
            pl.BlockSpec((BT, D), lambda i, be: (i, 0)),
            pl.BlockSpec((1, F, D), lambda i, be: (be[i], 0, 0)),
            pl.BlockSpec((1, F, D), lambda i, be: (be[i], 0, 0)),
            pl.BlockSpec((1, D, F), lambda i, be: (be[i], 0, 0)),
        ],
        out_specs=pl.BlockSpec((BT, D), lambda i, be: (i, 0)),
        scratch_shapes=[
            pltpu.VMEM((F, D), jnp.bfloat16),
            pltpu.VMEM((F, D), jnp.bfloat16),
            pltpu.VMEM((D, F), jnp.bfloat16),
            pltpu.SMEM((1,), jnp.int32),
        ],
    )
    return pl.pallas_call(
        _ffn_body,
        grid_spec=grid_spec,
        out_shape=jax.ShapeDtypeStruct((P, D), jnp.float32),
    )(be, xg, Wg, Wu, Wd)


@functools.partial(
    pl.kernel,
    out_type=jax.ShapeDtypeStruct((T, D), jnp.float32),
    mesh=_mesh,
    scratch_types=[
        pltpu.VMEM((TW,), jnp.int32),
        pltpu.VMEM((TW,), jnp.int32),
        pltpu.VMEM((TW,), jnp.float32),
        pltpu.VMEM((TW,), jnp.float32),
        pltpu.VMEM((CCH, D), jnp.float32),
        pltpu.VMEM((CCH, D), jnp.float32),
        pltpu.VMEM((CCH, D), jnp.float32),
        pltpu.VMEM((CCH, D), jnp.float32),
        pltpu.VMEM((CCH, D), jnp.float32),
        pltpu.VMEM((CCH, D), jnp.float32),
        pltpu.SemaphoreType.DMA,
        pltpu.SemaphoreType.DMA,
        pltpu.SemaphoreType.DMA,
    ],
    compiler_params=_sc_params,
)
def _combine(y_hbm, pos1_hbm, pos2_hbm, w1_hbm, w2_hbm, out_hbm,
             p1_v, p2_v, w1_v, w2_v, r1a_v, r2a_v, r1b_v, r2b_v,
             outa_v, outb_v, sem0, sem1, sem_o):
    cid = lax.axis_index("c")
    sid = lax.axis_index("s")
    wid = sid * NC + cid
    tb = wid * TW
    pltpu.sync_copy(pos1_hbm.at[pl.ds(tb, TW)], p1_v)
    pltpu.sync_copy(pos2_hbm.at[pl.ds(tb, TW)], p2_v)
    pltpu.sync_copy(w1_hbm.at[pl.ds(tb, TW)], w1_v)
    pltpu.sync_copy(w2_hbm.at[pl.ds(tb, TW)], w2_v)

    nch = TW // CCH
    r1 = [r1a_v, r1b_v]
    r2 = [r2a_v, r2b_v]
    sems = [sem0, sem1]

    def fire(ch, slot):
        a = pltpu.async_copy(y_hbm.at[p1_v.at[pl.ds(ch * CCH, CCH)]],
                             r1[slot], sems[slot])
        b = pltpu.async_copy(y_hbm.at[p2_v.at[pl.ds(ch * CCH, CCH)]],
                             r2[slot], sems[slot])
        return a, b

    outs = [outa_v, outb_v]
    pend = fire(0, 0)
    pend_out = [None, None]
    for ch in range(nch):
        slot = ch % 2
        pend[0].wait()
        pend[1].wait()
        if ch + 1 < nch:
            pend = fire(ch + 1, (ch + 1) % 2)
        if pend_out[slot] is not None:
            pend_out[slot].wait()
        wa = w1_v[pl.ds(ch * CCH, CCH)]
        wb = w2_v[pl.ds(ch * CCH, CCH)]
        out_v = outs[slot]

        def strip(j, _):
            for t in range(CCH):
                out_v[t, pl.ds(j * 16, 16)] = (
                    wa[t] * r1[slot][t, pl.ds(j * 16, 16)]
                    + wb[t] * r2[slot][t, pl.ds(j * 16, 16)])
            return 0
        lax.fori_loop(0, D // 16, strip, 0)
        pend_out[slot] = pltpu.async_copy(
            out_v, out_hbm.at[pl.ds(tb + ch * CCH, CCH)], sem_o)
    pend_out[0].wait()
    pend_out[1].wait()


def kernel(hidden_states, router_logits, Wg, Wu, Wd):
    logits_t = router_logits.T.reshape(E * T)
    xg, be, pos1, pos2, w1, w2 = _route_dispatch(logits_t, hidden_states)
    y = _expert_ffn(be, xg, Wg, Wu, Wd)
    return _combine(y, pos1, pos2, w1, w2)


# speedup vs baseline: 1.0810x; 1.0223x over previous
import functools

import jax
import jax.numpy as jnp
from jax import lax
from jax.experimental import pallas as pl
from jax.experimental.pallas import tpu as pltpu
from jax.experimental.pallas import tpu_sc as plsc

T = 2048
D = 1024
F = 512
E = 8
BT = 128
P = T * 2 + E * BT
NB = P // BT
NBPAD = 48

NC, NS = 2, 16
NW = NC * NS
TW = T // NW
NG = T // 16
GW = NG // NW
CCH = 16
NEG = -3.0e38

_mesh = plsc.VectorSubcoreMesh(core_axis_name="c", subcore_axis_name="s")
_sc_params = pltpu.CompilerParams(needs_layout_passes=False)


@functools.partial(
    pl.kernel,
    out_type=[
        jax.ShapeDtypeStruct((P, D), jnp.float32),
        jax.ShapeDtypeStruct((NBPAD,), jnp.int32),
        jax.ShapeDtypeStruct((T,), jnp.int32),
        jax.ShapeDtypeStruct((T,), jnp.int32),
        jax.ShapeDtypeStruct((T,), jnp.float32),
        jax.ShapeDtypeStruct((T,), jnp.float32),
    ],
    mesh=_mesh,
    scratch_types=[
        pltpu.VMEM((E * T,), jnp.float32),
        pltpu.VMEM((T,), jnp.int32),
        pltpu.VMEM((T,), jnp.int32),
        pltpu.VMEM((TW,), jnp.int32),
        pltpu.VMEM((TW,), jnp.int32),
        pltpu.VMEM((TW,), jnp.float32),
        pltpu.VMEM((TW,), jnp.float32),
        pltpu.VMEM((NBPAD,), jnp.int32),
        pltpu.VMEM((TW, D), jnp.float32),
        pltpu.SemaphoreType.DMA,
        pltpu.SemaphoreType.DMA,
    ],
    compiler_params=_sc_params,
)
def _route_dispatch(logits_hbm, hidden_hbm,
                    xg_hbm, be_hbm, pos1_hbm, pos2_hbm, w1_hbm, w2_hbm,
                    lt_v, e1_v, e2_v, p1s_v, p2s_v, w1s_v, w2s_v,
                    be_v, rows_v, sem, sem_h):
    cid = lax.axis_index("c")
    sid = lax.axis_index("s")
    wid = sid * NC + cid
    own_lo = wid * GW
    lanes = lax.broadcasted_iota(jnp.int32, (16,), 0)
    zeros16 = jnp.zeros((16,), jnp.int32)

    hid_cp = pltpu.async_copy(hidden_hbm.at[pl.ds(wid * TW, TW)], rows_v, sem_h)
    pltpu.sync_copy(logits_hbm, lt_v)

    def scan_a(g, carry):
        hist, pref = carry
        base = g * 16
        le = [lt_v[pl.ds(e * T + base, 16)] for e in range(E)]
        m1 = le[0]
        for e in range(1, E):
            m1 = jnp.maximum(m1, le[e])
        i1 = jnp.full((16,), -1, jnp.int32)
        for e in range(E):
            i1 = jnp.where((le[e] == m1) & (i1 < 0), e, i1)
        m2 = jnp.full((16,), NEG, jnp.float32)
        l2 = []
        for e in range(E):
            v = jnp.where(i1 == e, NEG, le[e])
            l2.append(v)
            m2 = jnp.maximum(m2, v)
        i2 = jnp.full((16,), -1, jnp.int32)
        for e in range(E):
            i2 = jnp.where((l2[e] == m2) & (i2 < 0), e, i2)
        e1_v[pl.ds(base, 16)] = i1
        e2_v[pl.ds(base, 16)] = i2
        delta = zeros16
        for e in range(E):
            c = (plsc.all_reduce_population_count(i1 == e)
                 + plsc.all_reduce_population_count(i2 == e))
            delta = delta + jnp.where(lanes == e, c, 0)
        hist = hist + delta
        pref = pref + jnp.where(g < own_lo, delta, zeros16)
        return hist, pref
    hist, pref = lax.fori_loop(0, NG, scan_a, (zeros16, zeros16))

    starts = []
    nexts = []
    acc = jnp.int32(0)
    for e in range(E):
        starts.append(acc)
        acc = (acc + hist[e] + (BT - 1)) & jnp.int32(~(BT - 1))
        nexts.append(acc)

    for vb in range(NBPAD // 16):
        bb = (vb * 16 + lanes) * BT
        bev = zeros16
        for e in range(E - 1):
            bev = bev + jnp.where(bb >= nexts[e], 1, 0)
        if vb == NBPAD // 16 - 1:
            bev = jnp.where(lanes == 15, acc, bev)
        be_v[pl.ds(vb * 16, 16)] = bev

    @pl.when(wid == 0)
    def _():
        pltpu.sync_copy(be_v, be_hbm)

    runs = [starts[e] + pref[e] + zeros16 for e in range(E)]
    for k in range(GW):
        base = (own_lo + k) * 16
        le = [lt_v[pl.ds(e * T + base, 16)] for e in range(E)]
        i1 = e1_v[pl.ds(base, 16)]
        i2 = e2_v[pl.ds(base, 16)]
        m1 = jnp.full((16,), NEG, jnp.float32)
        m2 = jnp.full((16,), NEG, jnp.float32)
        for e in range(E):
            m1 = jnp.where(i1 == e, le[e], m1)
            m2 = jnp.where(i2 == e, le[e], m2)
        wa = 1.0 / (1.0 + jnp.exp(m2 - m1))
        w1s_v[pl.ds(k * 16, 16)] = wa
        w2s_v[pl.ds(k * 16, 16)] = 1.0 - wa
        p1vec = zeros16
        p2vec = zeros16
        for e in range(E):
            m = i1 == e
            cs = plsc.cumsum(m.astype(jnp.int32))
            p1vec = jnp.where(m, runs[e] + cs - 1, p1vec)
            runs[e] = runs[e] + plsc.all_reduce_population_count(m)
            m = i2 == e
            cs = plsc.cumsum(m.astype(jnp.int32))
            p2vec = jnp.where(m, runs[e] + cs - 1, p2vec)
            runs[e] = runs[e] + plsc.all_reduce_population_count(m)
        p1s_v[pl.ds(k * 16, 16)] = p1vec
        p2s_v[pl.ds(k * 16, 16)] = p2vec

    tb = wid * TW
    pltpu.sync_copy(p1s_v, pos1_hbm.at[pl.ds(tb, TW)])
    pltpu.sync_copy(p2s_v, pos2_hbm.at[pl.ds(tb, TW)])
    pltpu.sync_copy(w1s_v, w1_hbm.at[pl.ds(tb, TW)])
    pltpu.sync_copy(w2s_v, w2_hbm.at[pl.ds(tb, TW)])

    hid_cp.wait()
    c1 = pltpu.async_copy(rows_v, xg_hbm.at[p1s_v], sem)
    c2 = pltpu.async_copy(rows_v, xg_hbm.at[p2s_v], sem)
    c1.wait()
    c2.wait()


def _ffn_body(be_ref, x_ref, wg_ref, wu_ref, wd_ref, y_ref,
              wgb_ref, wub_ref, wdb_ref, flag_ref):
    i = pl.program_id(0)
    e = be_ref[i]
    active = i * BT < be_ref[NBPAD - 1]

    @pl.when(active & ((i == 0) | (e != flag_ref[0])))
    def _():
        wgb_ref[...] = wg_ref[0].astype(jnp.bfloat16)
        wub_ref[...] = wu_ref[0].astype(jnp.bfloat16)
        wdb_ref[...] = wd_ref[0].astype(jnp.bfloat16)
        flag_ref[0] = e

    @pl.when(active)
    def _():
        x = x_ref[...].astype(jnp.bfloat16)
        g = lax.dot_general(x, wgb_ref[...], (((1,), (1,)), ((), ())),
                            preferred_element_type=jnp.float32)
        u = lax.dot_general(x, wub_ref[...], (((1,), (1,)), ((), ())),
                            preferred_element_type=jnp.float32)
        h = (g * jax.nn.sigmoid(g) * u).astype(jnp.bfloat16)
        y_ref[...] = lax.dot_general(
            h, wdb_ref[...], (((1,), (1,)), ((), ())),
            preferred_element_type=jnp.float32)


def _expert_ffn(be, xg, Wg, Wu, Wd):
    grid_spec = pltpu.PrefetchScalarGridSpec(
        num_scalar_prefetch=1,
        grid=(NB,),
        in_specs=[
            pl.BlockSpec((BT, D), lambda i, be: (i, 0)),
            pl.BlockSpec((1, F, D), lambda i, be: (be[i], 0, 0)),
            pl.BlockSpec((1, F, D), lambda i, be: (be[i], 0, 0)),
            pl.BlockSpec((1, D, F), lambda i, be: (be[i], 0, 0)),
        ],
        out_specs=pl.BlockSpec((BT, D), lambda i, be: (i, 0)),
        scratch_shapes=[
            pltpu.VMEM((F, D), jnp.bfloat16),
            pltpu.VMEM((F, D), jnp.bfloat16),
            pltpu.VMEM((D, F), jnp.bfloat16),
            pltpu.SMEM((1,), jnp.int32),
        ],
    )
    return pl.pallas_call(
        _ffn_body,
        grid_spec=grid_spec,
        out_shape=jax.ShapeDtypeStruct((P, D), jnp.float32),
    )(be, xg, Wg, Wu, Wd)


@functools.partial(
    pl.kernel,
    out_type=jax.ShapeDtypeStruct((T, D), jnp.float32),
    mesh=_mesh,
    scratch_types=[
        pltpu.VMEM((TW,), jnp.int32),
        pltpu.VMEM((TW,), jnp.int32),
        pltpu.VMEM((TW,), jnp.float32),
        pltpu.VMEM((TW,), jnp.float32),
        pltpu.VMEM((CCH, D), jnp.float32),
        pltpu.VMEM((CCH, D), jnp.float32),
        pltpu.VMEM((CCH, D), jnp.float32),
        pltpu.VMEM((CCH, D), jnp.float32),
        pltpu.VMEM((CCH, D), jnp.float32),
        pltpu.VMEM((CCH, D), jnp.float32),
        pltpu.SemaphoreType.DMA,
        pltpu.SemaphoreType.DMA,
        pltpu.SemaphoreType.DMA,
    ],
    compiler_params=_sc_params,
)
def _combine(y_hbm, pos1_hbm, pos2_hbm, w1_hbm, w2_hbm, out_hbm,
             p1_v, p2_v, w1_v, w2_v, r1a_v, r2a_v, r1b_v, r2b_v,
             outa_v, outb_v, sem0, sem1, sem_o):
    cid = lax.axis_index("c")
    sid = lax.axis_index("s")
    wid = sid * NC + cid
    tb = wid * TW
    pltpu.sync_copy(pos1_hbm.at[pl.ds(tb, TW)], p1_v)
    pltpu.sync_copy(pos2_hbm.at[pl.ds(tb, TW)], p2_v)
    pltpu.sync_copy(w1_hbm.at[pl.ds(tb, TW)], w1_v)
    pltpu.sync_copy(w2_hbm.at[pl.ds(tb, TW)], w2_v)

    nch = TW // CCH
    r1 = [r1a_v, r1b_v]
    r2 = [r2a_v, r2b_v]
    sems = [sem0, sem1]

    def fire(ch, slot):
        a = pltpu.async_copy(y_hbm.at[p1_v.at[pl.ds(ch * CCH, CCH)]],
                             r1[slot], sems[slot])
        b = pltpu.async_copy(y_hbm.at[p2_v.at[pl.ds(ch * CCH, CCH)]],
                             r2[slot], sems[slot])
        return a, b

    outs = [outa_v, outb_v]
    pend = fire(0, 0)
    pend_out = [None, None]
    for ch in range(nch):
        slot = ch % 2
        pend[0].wait()
        pend[1].wait()
        if ch + 1 < nch:
            pend = fire(ch + 1, (ch + 1) % 2)
        if pend_out[slot] is not None:
            pend_out[slot].wait()
        wa = w1_v[pl.ds(ch * CCH, CCH)]
        wb = w2_v[pl.ds(ch * CCH, CCH)]
        out_v = outs[slot]

        def strip(j, _):
            for t in range(CCH):
                out_v[t, pl.ds(j * 16, 16)] = (
                    wa[t] * r1[slot][t, pl.ds(j * 16, 16)]
                    + wb[t] * r2[slot][t, pl.ds(j * 16, 16)])
            return 0
        lax.fori_loop(0, D // 16, strip, 0)
        pend_out[slot] = pltpu.async_copy(
            out_v, out_hbm.at[pl.ds(tb + ch * CCH, CCH)], sem_o)
    pend_out[0].wait()
    pend_out[1].wait()


def kernel(hidden_states, router_logits, Wg, Wu, Wd):
    logits_t = router_logits.T.reshape(E * T)
    xg, be, pos1, pos2, w1, w2 = _route_dispatch(logits_t, hidden_states)
    y = _expert_ffn(be, xg, Wg, Wu, Wd)
    return _combine(y, pos1, pos2, w1, w2)


# speedup vs baseline: 1.2738x; 1.1784x over previous
import functools

import jax
import jax.numpy as jnp
from jax import lax
from jax.experimental import pallas as pl
from jax.experimental.pallas import tpu as pltpu
from jax.experimental.pallas import tpu_sc as plsc

T = 2048
D = 1024
F = 512
E = 8
BT = 256
P = T * 2 + E * BT
NB = P // BT
NBPAD = 32

NC, NS = 2, 16
NW = NC * NS
TW = T // NW
NG = T // 16
GW = NG // NW
CCH = 16
NEG = -3.0e38

_mesh = plsc.VectorSubcoreMesh(core_axis_name="c", subcore_axis_name="s")
_sc_params = pltpu.CompilerParams(needs_layout_passes=False)


@functools.partial(
    pl.kernel,
    out_type=[
        jax.ShapeDtypeStruct((P, D), jnp.float32),
        jax.ShapeDtypeStruct((NBPAD,), jnp.int32),
        jax.ShapeDtypeStruct((T,), jnp.int32),
        jax.ShapeDtypeStruct((T,), jnp.int32),
        jax.ShapeDtypeStruct((T,), jnp.float32),
        jax.ShapeDtypeStruct((T,), jnp.float32),
    ],
    mesh=_mesh,
    scratch_types=[
        pltpu.VMEM((E * T,), jnp.float32),
        pltpu.VMEM((T,), jnp.int32),
        pltpu.VMEM((T,), jnp.int32),
        pltpu.VMEM((TW,), jnp.int32),
        pltpu.VMEM((TW,), jnp.int32),
        pltpu.VMEM((TW,), jnp.float32),
        pltpu.VMEM((TW,), jnp.float32),
        pltpu.VMEM((NBPAD,), jnp.int32),
        pltpu.VMEM((TW, D), jnp.float32),
        pltpu.SemaphoreType.DMA,
        pltpu.SemaphoreType.DMA,
    ],
    compiler_params=_sc_params,
)
def _route_dispatch(logits_hbm, hidden_hbm,
                    xg_hbm, be_hbm, pos1_hbm, pos2_hbm, w1_hbm, w2_hbm,
                    lt_v, e1_v, e2_v, p1s_v, p2s_v, w1s_v, w2s_v,
                    be_v, rows_v, sem, sem_h):
    cid = lax.axis_index("c")
    sid = lax.axis_index("s")
    wid = sid * NC + cid
    own_lo = wid * GW
    lanes = lax.broadcasted_iota(jnp.int32, (16,), 0)
    zeros16 = jnp.zeros((16,), jnp.int32)

    hid_cp = pltpu.async_copy(hidden_hbm.at[pl.ds(wid * TW, TW)], rows_v, sem_h)
    pltpu.sync_copy(logits_hbm, lt_v)

    def scan_a(g, carry):
        hist, pref = carry
        base = g * 16
        le = [lt_v[pl.ds(e * T + base, 16)] for e in range(E)]
        m1 = le[0]
        for e in range(1, E):
            m1 = jnp.maximum(m1, le[e])
        i1 = jnp.full((16,), -1, jnp.int32)
        for e in range(E):
            i1 = jnp.where((le[e] == m1) & (i1 < 0), e, i1)
        m2 = jnp.full((16,), NEG, jnp.float32)
        l2 = []
        for e in range(E):
            v = jnp.where(i1 == e, NEG, le[e])
            l2.append(v)
            m2 = jnp.maximum(m2, v)
        i2 = jnp.full((16,), -1, jnp.int32)
        for e in range(E):
            i2 = jnp.where((l2[e] == m2) & (i2 < 0), e, i2)
        e1_v[pl.ds(base, 16)] = i1
        e2_v[pl.ds(base, 16)] = i2
        delta = zeros16
        for e in range(E):
            c = (plsc.all_reduce_population_count(i1 == e)
                 + plsc.all_reduce_population_count(i2 == e))
            delta = delta + jnp.where(lanes == e, c, 0)
        hist = hist + delta
        pref = pref + jnp.where(g < own_lo, delta, zeros16)
        return hist, pref
    hist, pref = lax.fori_loop(0, NG, scan_a, (zeros16, zeros16))

    starts = []
    nexts = []
    acc = jnp.int32(0)
    for e in range(E):
        starts.append(acc)
        acc = (acc + hist[e] + (BT - 1)) & jnp.int32(~(BT - 1))
        nexts.append(acc)

    for vb in range(NBPAD // 16):
        bb = (vb * 16 + lanes) * BT
        bev = zeros16
        for e in range(E - 1):
            bev = bev + jnp.where(bb >= nexts[e], 1, 0)
        if vb == NBPAD // 16 - 1:
            bev = jnp.where(lanes == 15, acc, bev)
        be_v[pl.ds(vb * 16, 16)] = bev

    @pl.when(wid == 0)
    def _():
        pltpu.sync_copy(be_v, be_hbm)

    runs = [starts[e] + pref[e] + zeros16 for e in range(E)]
    for k in range(GW):
        base = (own_lo + k) * 16
        le = [lt_v[pl.ds(e * T + base, 16)] for e in range(E)]
        i1 = e1_v[pl.ds(base, 16)]
        i2 = e2_v[pl.ds(base, 16)]
        m1 = jnp.full((16,), NEG, jnp.float32)
        m2 = jnp.full((16,), NEG, jnp.float32)
        for e in range(E):
            m1 = jnp.where(i1 == e, le[e], m1)
            m2 = jnp.where(i2 == e, le[e], m2)
        wa = 1.0 / (1.0 + jnp.exp(m2 - m1))
        w1s_v[pl.ds(k * 16, 16)] = wa
        w2s_v[pl.ds(k * 16, 16)] = 1.0 - wa
        p1vec = zeros16
        p2vec = zeros16
        for e in range(E):
            m = i1 == e
            cs = plsc.cumsum(m.astype(jnp.int32))
            p1vec = jnp.where(m, runs[e] + cs - 1, p1vec)
            runs[e] = runs[e] + plsc.all_reduce_population_count(m)
            m = i2 == e
            cs = plsc.cumsum(m.astype(jnp.int32))
            p2vec = jnp.where(m, runs[e] + cs - 1, p2vec)
            runs[e] = runs[e] + plsc.all_reduce_population_count(m)
        p1s_v[pl.ds(k * 16, 16)] = p1vec
        p2s_v[pl.ds(k * 16, 16)] = p2vec

    tb = wid * TW
    pltpu.sync_copy(p1s_v, pos1_hbm.at[pl.ds(tb, TW)])
    pltpu.sync_copy(p2s_v, pos2_hbm.at[pl.ds(tb, TW)])
    pltpu.sync_copy(w1s_v, w1_hbm.at[pl.ds(tb, TW)])
    pltpu.sync_copy(w2s_v, w2_hbm.at[pl.ds(tb, TW)])

    hid_cp.wait()
    c1 = pltpu.async_copy(rows_v, xg_hbm.at[p1s_v], sem)
    c2 = pltpu.async_copy(rows_v, xg_hbm.at[p2s_v], sem)
    c1.wait()
    c2.wait()


def _ffn_body(be_ref, x_ref, wg_ref, wu_ref, wd_ref, y_ref,
              wgb_ref, wub_ref, wdb_ref, flag_ref):
    i = pl.program_id(0)
    e = be_ref[i]
    active = i * BT < be_ref[NBPAD - 1]

    @pl.when(active & ((i == 0) | (e != flag_ref[0])))
    def _():
        wgb_ref[...] = wg_ref[0].astype(jnp.bfloat16)
        wub_ref[...] = wu_ref[0].astype(jnp.bfloat16)
        wdb_ref[...] = wd_ref[0].astype(jnp.bfloat16)
        flag_ref[0] = e

    @pl.when(active)
    def _():
        x = x_ref[...].astype(jnp.bfloat16)
        g = lax.dot_general(x, wgb_ref[...], (((1,), (1,)), ((), ())),
                            preferred_element_type=jnp.float32)
        u = lax.dot_general(x, wub_ref[...], (((1,), (1,)), ((), ())),
                            preferred_element_type=jnp.float32)
        h = (g * jax.nn.sigmoid(g) * u).astype(jnp.bfloat16)
        y_ref[...] = lax.dot_general(
            h, wdb_ref[...], (((1,), (1,)), ((), ())),
            preferred_element_type=jnp.float32)


def _expert_ffn(be, xg, Wg, Wu, Wd):
    grid_spec = pltpu.PrefetchScalarGridSpec(
        num_scalar_prefetch=1,
        grid=(NB,),
        in_specs=[
            pl.BlockSpec((BT, D), lambda i, be: (i, 0)),
            pl.BlockSpec((1, F, D), lambda i, be: (be[i], 0, 0)),
            pl.BlockSpec((1, F, D), lambda i, be: (be[i], 0, 0)),
            pl.BlockSpec((1, D, F), lambda i, be: (be[i], 0, 0)),
        ],
        out_specs=pl.BlockSpec((BT, D), lambda i, be: (i, 0)),
        scratch_shapes=[
            pltpu.VMEM((F, D), jnp.bfloat16),
            pltpu.VMEM((F, D), jnp.bfloat16),
            pltpu.VMEM((D, F), jnp.bfloat16),
            pltpu.SMEM((1,), jnp.int32),
        ],
    )
    return pl.pallas_call(
        _ffn_body,
        grid_spec=grid_spec,
        out_shape=jax.ShapeDtypeStruct((P, D), jnp.float32),
    )(be, xg, Wg, Wu, Wd)


@functools.partial(
    pl.kernel,
    out_type=jax.ShapeDtypeStruct((T, D), jnp.float32),
    mesh=_mesh,
    scratch_types=[
        pltpu.VMEM((TW,), jnp.int32),
        pltpu.VMEM((TW,), jnp.int32),
        pltpu.VMEM((TW,), jnp.float32),
        pltpu.VMEM((TW,), jnp.float32),
        pltpu.VMEM((CCH, D), jnp.float32),
        pltpu.VMEM((CCH, D), jnp.float32),
        pltpu.VMEM((CCH, D), jnp.float32),
        pltpu.VMEM((CCH, D), jnp.float32),
        pltpu.VMEM((CCH, D), jnp.float32),
        pltpu.VMEM((CCH, D), jnp.float32),
        pltpu.SemaphoreType.DMA,
        pltpu.SemaphoreType.DMA,
        pltpu.SemaphoreType.DMA,
    ],
    compiler_params=_sc_params,
)
def _combine(y_hbm, pos1_hbm, pos2_hbm, w1_hbm, w2_hbm, out_hbm,
             p1_v, p2_v, w1_v, w2_v, r1a_v, r2a_v, r1b_v, r2b_v,
             outa_v, outb_v, sem0, sem1, sem_o):
    cid = lax.axis_index("c")
    sid = lax.axis_index("s")
    wid = sid * NC + cid
    tb = wid * TW
    pltpu.sync_copy(pos1_hbm.at[pl.ds(tb, TW)], p1_v)
    pltpu.sync_copy(pos2_hbm.at[pl.ds(tb, TW)], p2_v)
    pltpu.sync_copy(w1_hbm.at[pl.ds(tb, TW)], w1_v)
    pltpu.sync_copy(w2_hbm.at[pl.ds(tb, TW)], w2_v)

    nch = TW // CCH
    r1 = [r1a_v, r1b_v]
    r2 = [r2a_v, r2b_v]
    sems = [sem0, sem1]

    def fire(ch, slot):
        a = pltpu.async_copy(y_hbm.at[p1_v.at[pl.ds(ch * CCH, CCH)]],
                             r1[slot], sems[slot])
        b = pltpu.async_copy(y_hbm.at[p2_v.at[pl.ds(ch * CCH, CCH)]],
                             r2[slot], sems[slot])
        return a, b

    outs = [outa_v, outb_v]
    pend = fire(0, 0)
    pend_out = [None, None]
    for ch in range(nch):
        slot = ch % 2
        pend[0].wait()
        pend[1].wait()
        if ch + 1 < nch:
            pend = fire(ch + 1, (ch + 1) % 2)
        if pend_out[slot] is not None:
            pend_out[slot].wait()
        wa = w1_v[pl.ds(ch * CCH, CCH)]
        wb = w2_v[pl.ds(ch * CCH, CCH)]
        out_v = outs[slot]

        def strip(j, _):
            for t in range(CCH):
                out_v[t, pl.ds(j * 16, 16)] = (
                    wa[t] * r1[slot][t, pl.ds(j * 16, 16)]
                    + wb[t] * r2[slot][t, pl.ds(j * 16, 16)])
            return 0
        lax.fori_loop(0, D // 16, strip, 0)
        pend_out[slot] = pltpu.async_copy(
            out_v, out_hbm.at[pl.ds(tb + ch * CCH, CCH)], sem_o)
    pend_out[0].wait()
    pend_out[1].wait()


def kernel(hidden_states, router_logits, Wg, Wu, Wd):
    logits_t = router_logits.T.reshape(E * T)
    xg, be, pos1, pos2, w1, w2 = _route_dispatch(logits_t, hidden_states)
    y = _expert_ffn(be, xg, Wg, Wu, Wd)
    return _combine(y, pos1, pos2, w1, w2)
